# baseline (speedup 1.0000x reference)
import functools

import jax
import jax.numpy as jnp
from jax import lax
from jax.experimental import pallas as pl
from jax.experimental.pallas import tpu as pltpu
from jax.experimental.pallas import tpu_sc as plsc

D = 128


@functools.cache
def _build(n_samples, seq):
    info = plsc.get_sparse_core_info()
    nw = info.num_cores * info.num_subcores
    per_w = n_samples // nw
    nb = 8
    k = 3

    mesh = plsc.VectorSubcoreMesh(core_axis_name="c", subcore_axis_name="s")

    @functools.partial(
        pl.kernel,
        mesh=mesh,
        compiler_params=pltpu.CompilerParams(use_tc_tiling_on_sc=True),
        out_type=jax.ShapeDtypeStruct((n_samples, seq, D), jnp.float32),
        scratch_types=[
            pltpu.VMEM((per_w, seq), jnp.int32),
            pltpu.VMEM((nb, seq, D), jnp.float32),
            pltpu.SemaphoreType.DMA((nb,)),
            pltpu.SemaphoreType.DMA((nb,)),
        ],
    )
    def gather_kernel(x_hbm, table_hbm, out_hbm, idx_v, rows_v, gsem, ssem):
        wid = lax.axis_index("s") * info.num_cores + lax.axis_index("c")
        s_base = wid * per_w
        pltpu.sync_copy(x_hbm.at[pl.ds(s_base, per_w)], idx_v)

        def buf(i):
            return i % nb if isinstance(i, int) else lax.rem(i, nb)

        def gather(i):
            b = buf(i)
            return pltpu.make_async_copy(
                table_hbm.at[idx_v.at[i]], rows_v.at[b], gsem.at[b]
            )

        def store(i):
            b = buf(i)
            return pltpu.make_async_copy(
                rows_v.at[b], out_hbm.at[s_base + i], ssem.at[b]
            )

        for i in range(nb):
            gather(i).start()
        for i in range(k):
            gather(i).wait()
            store(i).start()

        def body(i, carry):
            gather(i).wait()
            store(i).start()
            d = i - k
            store(d).wait()
            gather(d + nb).start()
            return carry

        lax.fori_loop(k, per_w - (nb - k), body, 0)

        for i in range(per_w - (nb - k), per_w):
            gather(i).wait()
            store(i).start()
        for i in range(per_w - nb, per_w):
            store(i).wait()

    return gather_kernel


def kernel(x, table):
    n_samples, seq = x.shape
    return _build(n_samples, seq)(x.astype(jnp.int32), table)


# speedup vs baseline: 1.7927x; 1.7927x over previous
import functools

import jax
import jax.numpy as jnp
from jax import lax
from jax.experimental import pallas as pl
from jax.experimental.pallas import tpu as pltpu
from jax.experimental.pallas import tpu_sc as plsc

D = 128


@functools.cache
def _build(seq, n_samples):
    info = plsc.get_sparse_core_info()
    nw = info.num_cores * info.num_subcores
    per_w = n_samples // nw
    nb = 5
    k = 2

    mesh = plsc.VectorSubcoreMesh(core_axis_name="c", subcore_axis_name="s")

    @functools.partial(
        pl.kernel,
        mesh=mesh,
        out_type=jax.ShapeDtypeStruct((seq, n_samples, D), jnp.float32),
        scratch_types=[
            pltpu.VMEM((seq, per_w), jnp.int32),
            pltpu.VMEM((nb, per_w, D), jnp.float32),
            pltpu.SemaphoreType.DMA((nb,)),
            pltpu.SemaphoreType.DMA((nb,)),
        ],
    )
    def gather_kernel(xt_hbm, table_hbm, out_hbm, idx_v, rows_v, gsem, ssem):
        wid = lax.axis_index("s") * info.num_cores + lax.axis_index("c")
        s0 = wid * per_w
        pltpu.sync_copy(xt_hbm.at[:, pl.ds(s0, per_w)], idx_v)

        def buf(t):
            return t % nb if isinstance(t, int) else lax.rem(t, nb)

        def gather(t):
            b = buf(t)
            return pltpu.make_async_copy(
                table_hbm.at[idx_v.at[t]], rows_v.at[b], gsem.at[b]
            )

        def store(t):
            b = buf(t)
            return pltpu.make_async_copy(
                rows_v.at[b], out_hbm.at[t, pl.ds(s0, per_w)], ssem.at[b]
            )

        for t in range(nb):
            gather(t).start()
        for t in range(k):
            gather(t).wait()
            store(t).start()

        def body(t, carry):
            gather(t).wait()
            store(t).start()
            d = t - k
            store(d).wait()
            gather(d + nb).start()
            return carry

        lax.fori_loop(k, seq - (nb - k), body, 0)

        for t in range(seq - (nb - k), seq):
            gather(t).wait()
            store(t).start()
        for t in range(seq - nb, seq):
            store(t).wait()

    return gather_kernel


def kernel(x, table):
    n_samples, seq = x.shape
    xt = x.T.astype(jnp.int32)
    out = _build(seq, n_samples)(xt, table)
    return jnp.transpose(out, (1, 0, 2))


# speedup vs baseline: 1.8045x; 1.0066x over previous
import functools

import jax
import jax.numpy as jnp
from jax import lax
from jax.experimental import pallas as pl
from jax.experimental.pallas import tpu as pltpu
from jax.experimental.pallas import tpu_sc as plsc

D = 128


@functools.cache
def _build(seq, n_samples):
    info = plsc.get_sparse_core_info()
    nw = info.num_cores * info.num_subcores
    per_w = n_samples // nw
    nb = 7
    k = 3

    mesh = plsc.VectorSubcoreMesh(core_axis_name="c", subcore_axis_name="s")

    @functools.partial(
        pl.kernel,
        mesh=mesh,
        out_type=jax.ShapeDtypeStruct((seq, n_samples, D), jnp.float32),
        scratch_types=[
            pltpu.VMEM((seq, per_w), jnp.int32),
            pltpu.VMEM((nb, per_w, D), jnp.float32),
            pltpu.SemaphoreType.DMA((nb,)),
            pltpu.SemaphoreType.DMA((nb,)),
        ],
    )
    def gather_kernel(xt_hbm, table_hbm, out_hbm, idx_v, rows_v, gsem, ssem):
        wid = lax.axis_index("s") * info.num_cores + lax.axis_index("c")
        s0 = wid * per_w
        pltpu.sync_copy(xt_hbm.at[:, pl.ds(s0, per_w)], idx_v)

        def buf(t):
            return t % nb if isinstance(t, int) else lax.rem(t, nb)

        def gather(t):
            b = buf(t)
            return pltpu.make_async_copy(
                table_hbm.at[idx_v.at[t]], rows_v.at[b], gsem.at[b]
            )

        def store(t):
            b = buf(t)
            return pltpu.make_async_copy(
                rows_v.at[b], out_hbm.at[t, pl.ds(s0, per_w)], ssem.at[b]
            )

        for t in range(nb):
            gather(t).start()
        for t in range(k):
            gather(t).wait()
            store(t).start()

        def body(t, carry):
            gather(t).wait()
            store(t).start()
            d = t - k
            store(d).wait()
            gather(d + nb).start()
            return carry

        lax.fori_loop(k, seq - (nb - k), body, 0)

        for t in range(seq - (nb - k), seq):
            gather(t).wait()
            store(t).start()
        for t in range(seq - nb, seq):
            store(t).wait()

    return gather_kernel


def kernel(x, table):
    n_samples, seq = x.shape
    xt = x.T.astype(jnp.int32)
    out = _build(seq, n_samples)(xt, table)
    return jnp.transpose(out, (1, 0, 2))
